# Initial kernel scaffold; baseline (speedup 1.0000x reference)
import jax
import jax.numpy as jnp
from jax.experimental import pallas as pl


def kernel(x, edge_index, Wg, bg, W1, b1, W3, b3):
    raise NotImplementedError("write your pallas kernel here")


# speedup vs baseline: 60.2571x; 60.2571x over previous
import functools

import jax
import jax.numpy as jnp
from jax import lax
from jax.experimental import pallas as pl
from jax.experimental.pallas import tpu as pltpu
from jax.experimental.pallas import tpu_sc as plsc

N_NODES = 100000
N_PAD = 102400
N_EDGES = 6400000
EROWS = N_EDGES // 128
CHUNK_ROWS = 8
N_CHUNKS = EROWS // CHUNK_ROWS
NW = 32
TILE_ITERS = -(-N_CHUNKS // NW)
D = 8
TC_BLK = 4096
TC_GRID = N_PAD // TC_BLK


def _sc_degree_body(dst_hbm, zeros_hbm, ones_hbm, deg_out, idx_v, ones_v, deg_sh, sem):
    cid = lax.axis_index("c")
    sid = lax.axis_index("s")
    wid = sid * 2 + cid

    @pl.when(sid == 0)
    def _():
        pltpu.sync_copy(zeros_hbm, deg_sh)

    pltpu.sync_copy(ones_hbm, ones_v)
    plsc.subcore_barrier()

    def body(i, carry):
        c = wid + i * NW

        @pl.when(c < N_CHUNKS)
        def _():
            pltpu.sync_copy(dst_hbm.at[pl.ds(c * CHUNK_ROWS, CHUNK_ROWS)], idx_v)
            for j in range(CHUNK_ROWS):
                pltpu.sync_copy(ones_v, deg_sh.at[idx_v.at[j]], add=True)

        return carry

    lax.fori_loop(0, TILE_ITERS, body, 0)
    plsc.subcore_barrier()
    rows_per_tile = N_PAD // 16
    pltpu.sync_copy(
        deg_sh.at[pl.ds(sid * rows_per_tile, rows_per_tile)],
        deg_out.at[cid, pl.ds(sid * rows_per_tile, rows_per_tile)],
    )


def _sc_message_body(src_hbm, dst_hbm, y_hbm, zeros_hbm, acc_out,
                     idx_s, idx_d, rows_v, acc_sh, sem):
    cid = lax.axis_index("c")
    sid = lax.axis_index("s")
    wid = sid * 2 + cid

    @pl.when(sid == 0)
    def _():
        pltpu.sync_copy(zeros_hbm, acc_sh)

    plsc.subcore_barrier()

    def body(i, carry):
        c = wid + i * NW

        @pl.when(c < N_CHUNKS)
        def _():
            pltpu.sync_copy(src_hbm.at[pl.ds(c * CHUNK_ROWS, CHUNK_ROWS)], idx_s)
            pltpu.sync_copy(dst_hbm.at[pl.ds(c * CHUNK_ROWS, CHUNK_ROWS)], idx_d)
            for j in range(CHUNK_ROWS):
                pltpu.async_copy(y_hbm.at[idx_s.at[j]], rows_v, sem).wait()
                pltpu.sync_copy(rows_v, acc_sh.at[idx_d.at[j]], add=True)

        return carry

    lax.fori_loop(0, TILE_ITERS, body, 0)
    plsc.subcore_barrier()
    rows_per_tile = N_PAD // 16
    pltpu.sync_copy(
        acc_sh.at[pl.ds(sid * rows_per_tile, rows_per_tile)],
        acc_out.at[cid, pl.ds(sid * rows_per_tile, rows_per_tile)],
    )


@functools.cache
def _build_sc_kernels():
    mesh = plsc.VectorSubcoreMesh(core_axis_name="c", subcore_axis_name="s")
    params = pltpu.CompilerParams(use_tc_tiling_on_sc=False)
    deg = functools.partial(
        pl.kernel,
        out_type=jax.ShapeDtypeStruct((2, N_PAD, D), jnp.float32),
        mesh=mesh,
        compiler_params=params,
        scratch_types=[
            pltpu.VMEM((CHUNK_ROWS, 128), jnp.int32),
            pltpu.VMEM((128, D), jnp.float32),
            pltpu.VMEM_SHARED((N_PAD, D), jnp.float32),
            pltpu.SemaphoreType.DMA,
        ],
    )(_sc_degree_body)
    msg = functools.partial(
        pl.kernel,
        out_type=jax.ShapeDtypeStruct((2, N_PAD, D), jnp.float32),
        mesh=mesh,
        compiler_params=params,
        scratch_types=[
            pltpu.VMEM((CHUNK_ROWS, 128), jnp.int32),
            pltpu.VMEM((CHUNK_ROWS, 128), jnp.int32),
            pltpu.VMEM((128, D), jnp.float32),
            pltpu.VMEM_SHARED((N_PAD, D), jnp.float32),
            pltpu.SemaphoreType.DMA,
        ],
    )(_sc_message_body)
    return deg, msg


def _tc_normalize_body(deg_ref, x_ref, wg_ref, y_ref):
    deg = deg_ref[0] + deg_ref[1] + 1.0
    dinv = lax.rsqrt(deg)
    x = x_ref[...]
    wg = wg_ref[...]
    xw = x[:, 0:1] * wg[0:1, :]
    for k in range(1, 6):
        xw = xw + x[:, k : k + 1] * wg[k : k + 1, :]
    y_ref[...] = xw * dinv


def _tc_finalize_body(acc_ref, y_ref, deg_ref, bg_ref, w1_ref, b1_ref,
                      w3_ref, b3_ref, out_ref, sum_ref):
    i = pl.program_id(0)
    deg = deg_ref[0] + deg_ref[1] + 1.0
    dinv = lax.rsqrt(deg)
    pre = dinv * (acc_ref[0] + acc_ref[1] + y_ref[...]) + bg_ref[...]
    h = jnp.tanh(pre)
    rows = i * TC_BLK + lax.broadcasted_iota(jnp.int32, (TC_BLK, 1), 0)
    h = jnp.where(rows < N_NODES, h, 0.0)
    psum = jnp.sum(h, axis=0, keepdims=True)

    @pl.when(i == 0)
    def _():
        sum_ref[0:1, 0:D] = psum

    @pl.when(i > 0)
    def _():
        sum_ref[0:1, 0:D] = sum_ref[0:1, 0:D] + psum

    @pl.when(i == TC_GRID - 1)
    def _():
        g = sum_ref[0:1, 0:D] * (1.0 / N_NODES)
        r = jnp.dot(g, w1_ref[...], preferred_element_type=jnp.float32,
                    precision=lax.Precision.HIGHEST)
        r = jnp.maximum(r + b1_ref[...], 0.0)
        o = jnp.dot(r, w3_ref[...], preferred_element_type=jnp.float32,
                    precision=lax.Precision.HIGHEST)
        out_ref[...] = o + b3_ref[...]


def kernel(x, edge_index, Wg, bg, W1, b1, W3, b3):
    ei = edge_index.astype(jnp.int32).reshape(2, EROWS, 128)
    src2d = ei[0]
    dst2d = ei[1]
    zeros = jnp.zeros((N_PAD, D), jnp.float32)
    ones = jnp.ones((128, D), jnp.float32)
    x_pad = jnp.pad(x, ((0, N_PAD - N_NODES), (0, 0)))
    wg8 = jnp.pad(Wg, ((0, 0), (0, D - 4)))
    bg8 = jnp.pad(bg.reshape(1, 4), ((0, 0), (0, D - 4)))
    w18 = jnp.pad(W1, ((0, D - 4), (0, 0)))

    sc_degree, sc_message = _build_sc_kernels()
    deg_parts = sc_degree(dst2d, zeros, ones)

    y = pl.pallas_call(
        _tc_normalize_body,
        grid=(TC_GRID,),
        in_specs=[
            pl.BlockSpec((2, TC_BLK, D), lambda i: (0, i, 0)),
            pl.BlockSpec((TC_BLK, 6), lambda i: (i, 0)),
            pl.BlockSpec((6, D), lambda i: (0, 0)),
        ],
        out_specs=pl.BlockSpec((TC_BLK, D), lambda i: (i, 0)),
        out_shape=jax.ShapeDtypeStruct((N_PAD, D), jnp.float32),
    )(deg_parts, x_pad, wg8)

    acc_parts = sc_message(src2d, dst2d, y, zeros)

    out = pl.pallas_call(
        _tc_finalize_body,
        grid=(TC_GRID,),
        in_specs=[
            pl.BlockSpec((2, TC_BLK, D), lambda i: (0, i, 0)),
            pl.BlockSpec((TC_BLK, D), lambda i: (i, 0)),
            pl.BlockSpec((2, TC_BLK, D), lambda i: (0, i, 0)),
            pl.BlockSpec((1, D), lambda i: (0, 0)),
            pl.BlockSpec((D, 256), lambda i: (0, 0)),
            pl.BlockSpec((1, 256), lambda i: (0, 0)),
            pl.BlockSpec((256, 128), lambda i: (0, 0)),
            pl.BlockSpec((1, 128), lambda i: (0, 0)),
        ],
        out_specs=pl.BlockSpec((1, 128), lambda i: (0, 0)),
        out_shape=jax.ShapeDtypeStruct((1, 128), jnp.float32),
        scratch_shapes=[pltpu.VMEM((8, 128), jnp.float32)],
    )(acc_parts, y, deg_parts, bg8, w18, b1.reshape(1, 256),
      W3, b3.reshape(1, 128))

    return out


# speedup vs baseline: 139.8997x; 2.3217x over previous
import functools

import jax
import jax.numpy as jnp
from jax import lax
from jax.experimental import pallas as pl
from jax.experimental.pallas import tpu as pltpu
from jax.experimental.pallas import tpu_sc as plsc

N_NODES = 100000
N_PAD = 102400
N_EDGES = 6400000
EROWS = N_EDGES // 128
CHUNK_ROWS = 16
N_CHUNKS = EROWS // CHUNK_ROWS
NW = 32
TILE_ITERS = -(-N_CHUNKS // NW)
D = 8
TC_BLK = 4096
TC_GRID = N_PAD // TC_BLK


def _sc_degree_body(dst_hbm, zeros_hbm, ones_hbm, deg_out, idx_v, ones_v, deg_sh, sem):
    cid = lax.axis_index("c")
    sid = lax.axis_index("s")
    wid = sid * 2 + cid

    @pl.when(sid == 0)
    def _():
        pltpu.sync_copy(zeros_hbm, deg_sh)

    pltpu.sync_copy(ones_hbm, ones_v)
    plsc.subcore_barrier()

    def body(i, carry):
        c = wid + i * NW

        @pl.when(c < N_CHUNKS)
        def _():
            pltpu.sync_copy(dst_hbm.at[pl.ds(c * CHUNK_ROWS, CHUNK_ROWS)], idx_v)
            handles = [
                pltpu.async_copy(ones_v, deg_sh.at[idx_v.at[j]], sem, add=True)
                for j in range(CHUNK_ROWS)
            ]
            for h in handles:
                h.wait()

        return carry

    lax.fori_loop(0, TILE_ITERS, body, 0)
    plsc.subcore_barrier()
    rows_per_tile = N_PAD // 16
    pltpu.sync_copy(
        deg_sh.at[pl.ds(sid * rows_per_tile, rows_per_tile)],
        deg_out.at[cid, pl.ds(sid * rows_per_tile, rows_per_tile)],
    )


def _sc_message_body(src_hbm, dst_hbm, y_hbm, zeros_hbm, acc_out,
                     idx_s, idx_d, rows_v, acc_sh, gsem, ssem):
    cid = lax.axis_index("c")
    sid = lax.axis_index("s")
    wid = sid * 2 + cid

    @pl.when(sid == 0)
    def _():
        pltpu.sync_copy(zeros_hbm, acc_sh)

    plsc.subcore_barrier()

    def body(i, carry):
        c = wid + i * NW

        @pl.when(c < N_CHUNKS)
        def _():
            pltpu.sync_copy(src_hbm.at[pl.ds(c * CHUNK_ROWS, CHUNK_ROWS)], idx_s)
            pltpu.sync_copy(dst_hbm.at[pl.ds(c * CHUNK_ROWS, CHUNK_ROWS)], idx_d)
            gathers = [
                pltpu.async_copy(y_hbm.at[idx_s.at[j]], rows_v.at[j], gsem.at[j])
                for j in range(CHUNK_ROWS)
            ]
            scatters = []
            for j in range(CHUNK_ROWS):
                gathers[j].wait()
                scatters.append(
                    pltpu.async_copy(rows_v.at[j], acc_sh.at[idx_d.at[j]], ssem, add=True)
                )
            for h in scatters:
                h.wait()

        return carry

    lax.fori_loop(0, TILE_ITERS, body, 0)
    plsc.subcore_barrier()
    rows_per_tile = N_PAD // 16
    pltpu.sync_copy(
        acc_sh.at[pl.ds(sid * rows_per_tile, rows_per_tile)],
        acc_out.at[cid, pl.ds(sid * rows_per_tile, rows_per_tile)],
    )


@functools.cache
def _build_sc_kernels():
    mesh = plsc.VectorSubcoreMesh(core_axis_name="c", subcore_axis_name="s")
    params = pltpu.CompilerParams(use_tc_tiling_on_sc=False)
    deg = functools.partial(
        pl.kernel,
        out_type=jax.ShapeDtypeStruct((2, N_PAD, D), jnp.float32),
        mesh=mesh,
        compiler_params=params,
        scratch_types=[
            pltpu.VMEM((CHUNK_ROWS, 128), jnp.int32),
            pltpu.VMEM((128, D), jnp.float32),
            pltpu.VMEM_SHARED((N_PAD, D), jnp.float32),
            pltpu.SemaphoreType.DMA,
        ],
    )(_sc_degree_body)
    msg = functools.partial(
        pl.kernel,
        out_type=jax.ShapeDtypeStruct((2, N_PAD, D), jnp.float32),
        mesh=mesh,
        compiler_params=params,
        scratch_types=[
            pltpu.VMEM((CHUNK_ROWS, 128), jnp.int32),
            pltpu.VMEM((CHUNK_ROWS, 128), jnp.int32),
            pltpu.VMEM((CHUNK_ROWS, 128, D), jnp.float32),
            pltpu.VMEM_SHARED((N_PAD, D), jnp.float32),
            pltpu.SemaphoreType.DMA((CHUNK_ROWS,)),
            pltpu.SemaphoreType.DMA,
        ],
    )(_sc_message_body)
    return deg, msg


def _tc_normalize_body(deg_ref, x_ref, wg_ref, y_ref):
    deg = deg_ref[0] + deg_ref[1] + 1.0
    dinv = lax.rsqrt(deg)
    x = x_ref[...]
    wg = wg_ref[...]
    xw = x[:, 0:1] * wg[0:1, :]
    for k in range(1, 6):
        xw = xw + x[:, k : k + 1] * wg[k : k + 1, :]
    y_ref[...] = xw * dinv


def _tc_finalize_body(acc_ref, y_ref, deg_ref, bg_ref, w1_ref, b1_ref,
                      w3_ref, b3_ref, out_ref, sum_ref):
    i = pl.program_id(0)
    deg = deg_ref[0] + deg_ref[1] + 1.0
    dinv = lax.rsqrt(deg)
    pre = dinv * (acc_ref[0] + acc_ref[1] + y_ref[...]) + bg_ref[...]
    h = jnp.tanh(pre)
    rows = i * TC_BLK + lax.broadcasted_iota(jnp.int32, (TC_BLK, 1), 0)
    h = jnp.where(rows < N_NODES, h, 0.0)
    psum = jnp.sum(h, axis=0, keepdims=True)

    @pl.when(i == 0)
    def _():
        sum_ref[0:1, 0:D] = psum

    @pl.when(i > 0)
    def _():
        sum_ref[0:1, 0:D] = sum_ref[0:1, 0:D] + psum

    @pl.when(i == TC_GRID - 1)
    def _():
        g = sum_ref[0:1, 0:D] * (1.0 / N_NODES)
        r = jnp.dot(g, w1_ref[...], preferred_element_type=jnp.float32,
                    precision=lax.Precision.HIGHEST)
        r = jnp.maximum(r + b1_ref[...], 0.0)
        o = jnp.dot(r, w3_ref[...], preferred_element_type=jnp.float32,
                    precision=lax.Precision.HIGHEST)
        out_ref[...] = o + b3_ref[...]


def kernel(x, edge_index, Wg, bg, W1, b1, W3, b3):
    ei = edge_index.astype(jnp.int32).reshape(2, EROWS, 128)
    src2d = ei[0]
    dst2d = ei[1]
    zeros = jnp.zeros((N_PAD, D), jnp.float32)
    ones = jnp.ones((128, D), jnp.float32)
    x_pad = jnp.pad(x, ((0, N_PAD - N_NODES), (0, 0)))
    wg8 = jnp.pad(Wg, ((0, 0), (0, D - 4)))
    bg8 = jnp.pad(bg.reshape(1, 4), ((0, 0), (0, D - 4)))
    w18 = jnp.pad(W1, ((0, D - 4), (0, 0)))

    sc_degree, sc_message = _build_sc_kernels()
    deg_parts = sc_degree(dst2d, zeros, ones)

    y = pl.pallas_call(
        _tc_normalize_body,
        grid=(TC_GRID,),
        in_specs=[
            pl.BlockSpec((2, TC_BLK, D), lambda i: (0, i, 0)),
            pl.BlockSpec((TC_BLK, 6), lambda i: (i, 0)),
            pl.BlockSpec((6, D), lambda i: (0, 0)),
        ],
        out_specs=pl.BlockSpec((TC_BLK, D), lambda i: (i, 0)),
        out_shape=jax.ShapeDtypeStruct((N_PAD, D), jnp.float32),
    )(deg_parts, x_pad, wg8)

    acc_parts = sc_message(src2d, dst2d, y, zeros)

    out = pl.pallas_call(
        _tc_finalize_body,
        grid=(TC_GRID,),
        in_specs=[
            pl.BlockSpec((2, TC_BLK, D), lambda i: (0, i, 0)),
            pl.BlockSpec((TC_BLK, D), lambda i: (i, 0)),
            pl.BlockSpec((2, TC_BLK, D), lambda i: (0, i, 0)),
            pl.BlockSpec((1, D), lambda i: (0, 0)),
            pl.BlockSpec((D, 256), lambda i: (0, 0)),
            pl.BlockSpec((1, 256), lambda i: (0, 0)),
            pl.BlockSpec((256, 128), lambda i: (0, 0)),
            pl.BlockSpec((1, 128), lambda i: (0, 0)),
        ],
        out_specs=pl.BlockSpec((1, 128), lambda i: (0, 0)),
        out_shape=jax.ShapeDtypeStruct((1, 128), jnp.float32),
        scratch_shapes=[pltpu.VMEM((8, 128), jnp.float32)],
    )(acc_parts, y, deg_parts, bg8, w18, b1.reshape(1, 256),
      W3, b3.reshape(1, 128))

    return out


# speedup vs baseline: 143.6979x; 1.0271x over previous
import functools

import jax
import jax.numpy as jnp
from jax import lax
from jax.experimental import pallas as pl
from jax.experimental.pallas import tpu as pltpu
from jax.experimental.pallas import tpu_sc as plsc

N_NODES = 100000
N_PAD = 102400
N_EDGES = 6400000
EROWS = N_EDGES // 128
CHUNK_ROWS = 16
N_CHUNKS = EROWS // CHUNK_ROWS
NW = 32
TILE_ITERS = -(-N_CHUNKS // NW)
D = 8
TC_BLK = 4096
TC_GRID = N_PAD // TC_BLK


def _sc_degree_body(edge_hbm, zeros_hbm, ones_hbm, deg_out, idx_v, ones_v, deg_sh, sem):
    cid = lax.axis_index("c")
    sid = lax.axis_index("s")
    wid = sid * 2 + cid

    @pl.when(sid == 0)
    def _():
        pltpu.sync_copy(zeros_hbm, deg_sh)

    pltpu.sync_copy(ones_hbm, ones_v)
    plsc.subcore_barrier()

    def body(i, carry):
        c = wid + i * NW

        @pl.when(c < N_CHUNKS)
        def _():
            pltpu.sync_copy(edge_hbm.at[1, pl.ds(c * CHUNK_ROWS, CHUNK_ROWS)], idx_v)
            handles = [
                pltpu.async_copy(ones_v, deg_sh.at[idx_v.at[j]], sem, add=True)
                for j in range(CHUNK_ROWS)
            ]
            for h in handles:
                h.wait()

        return carry

    lax.fori_loop(0, TILE_ITERS, body, 0)
    plsc.subcore_barrier()
    rows_per_tile = N_PAD // 16
    pltpu.sync_copy(
        deg_sh.at[pl.ds(sid * rows_per_tile, rows_per_tile)],
        deg_out.at[cid, pl.ds(sid * rows_per_tile, rows_per_tile)],
    )


def _sc_message_body(edge_hbm, y_hbm, zeros_hbm, acc_out,
                     idx_s, idx_d, rows_v, acc_sh, gsem, ssem):
    cid = lax.axis_index("c")
    sid = lax.axis_index("s")
    wid = sid * 2 + cid

    @pl.when(sid == 0)
    def _():
        pltpu.sync_copy(zeros_hbm, acc_sh)

    plsc.subcore_barrier()

    def body(i, carry):
        c = wid + i * NW

        @pl.when(c < N_CHUNKS)
        def _():
            pltpu.sync_copy(edge_hbm.at[0, pl.ds(c * CHUNK_ROWS, CHUNK_ROWS)], idx_s)
            pltpu.sync_copy(edge_hbm.at[1, pl.ds(c * CHUNK_ROWS, CHUNK_ROWS)], idx_d)
            gathers = [
                pltpu.async_copy(y_hbm.at[idx_s.at[j]], rows_v.at[j], gsem.at[j])
                for j in range(CHUNK_ROWS)
            ]
            scatters = []
            for j in range(CHUNK_ROWS):
                gathers[j].wait()
                scatters.append(
                    pltpu.async_copy(rows_v.at[j], acc_sh.at[idx_d.at[j]], ssem, add=True)
                )
            for h in scatters:
                h.wait()

        return carry

    lax.fori_loop(0, TILE_ITERS, body, 0)
    plsc.subcore_barrier()
    rows_per_tile = N_PAD // 16
    pltpu.sync_copy(
        acc_sh.at[pl.ds(sid * rows_per_tile, rows_per_tile)],
        acc_out.at[cid, pl.ds(sid * rows_per_tile, rows_per_tile)],
    )


@functools.cache
def _build_sc_kernels():
    mesh = plsc.VectorSubcoreMesh(core_axis_name="c", subcore_axis_name="s")
    params = pltpu.CompilerParams(use_tc_tiling_on_sc=False)
    deg = functools.partial(
        pl.kernel,
        out_type=jax.ShapeDtypeStruct((2, N_PAD, D), jnp.float32),
        mesh=mesh,
        compiler_params=params,
        scratch_types=[
            pltpu.VMEM((CHUNK_ROWS, 128), jnp.int32),
            pltpu.VMEM((128, D), jnp.float32),
            pltpu.VMEM_SHARED((N_PAD, D), jnp.float32),
            pltpu.SemaphoreType.DMA,
        ],
    )(_sc_degree_body)
    msg = functools.partial(
        pl.kernel,
        out_type=jax.ShapeDtypeStruct((2, N_PAD, D), jnp.float32),
        mesh=mesh,
        compiler_params=params,
        scratch_types=[
            pltpu.VMEM((CHUNK_ROWS, 128), jnp.int32),
            pltpu.VMEM((CHUNK_ROWS, 128), jnp.int32),
            pltpu.VMEM((CHUNK_ROWS, 128, D), jnp.float32),
            pltpu.VMEM_SHARED((N_PAD, D), jnp.float32),
            pltpu.SemaphoreType.DMA((CHUNK_ROWS,)),
            pltpu.SemaphoreType.DMA,
        ],
    )(_sc_message_body)
    return deg, msg


def _tc_normalize_body(deg_ref, x_ref, wg_ref, y_ref):
    deg = deg_ref[0] + deg_ref[1] + 1.0
    dinv = lax.rsqrt(deg)
    x = x_ref[...]
    wg = wg_ref[...]
    xw = x[:, 0:1] * wg[0:1, :]
    for k in range(1, 6):
        xw = xw + x[:, k : k + 1] * wg[k : k + 1, :]
    y_ref[...] = xw * dinv


def _tc_finalize_body(acc_ref, y_ref, deg_ref, bg_ref, w1_ref, b1_ref,
                      w3_ref, b3_ref, out_ref, sum_ref):
    i = pl.program_id(0)
    deg = deg_ref[0] + deg_ref[1] + 1.0
    dinv = lax.rsqrt(deg)
    pre = dinv * (acc_ref[0] + acc_ref[1] + y_ref[...]) + bg_ref[...]
    h = jnp.tanh(pre)
    rows = i * TC_BLK + lax.broadcasted_iota(jnp.int32, (TC_BLK, 1), 0)
    h = jnp.where(rows < N_NODES, h, 0.0)
    psum = jnp.sum(h, axis=0, keepdims=True)

    @pl.when(i == 0)
    def _():
        sum_ref[0:1, 0:D] = psum

    @pl.when(i > 0)
    def _():
        sum_ref[0:1, 0:D] = sum_ref[0:1, 0:D] + psum

    @pl.when(i == TC_GRID - 1)
    def _():
        g = sum_ref[0:1, 0:D] * (1.0 / N_NODES)
        r = jnp.dot(g, w1_ref[...], preferred_element_type=jnp.float32,
                    precision=lax.Precision.HIGHEST)
        r = jnp.maximum(r + b1_ref[...], 0.0)
        o = jnp.dot(r, w3_ref[...], preferred_element_type=jnp.float32,
                    precision=lax.Precision.HIGHEST)
        out_ref[...] = o + b3_ref[...]


def kernel(x, edge_index, Wg, bg, W1, b1, W3, b3):
    ei = edge_index.astype(jnp.int32).reshape(2, EROWS, 128)
    zeros = jnp.zeros((N_PAD, D), jnp.float32)
    ones = jnp.ones((128, D), jnp.float32)
    x_pad = jnp.pad(x, ((0, N_PAD - N_NODES), (0, 0)))
    wg8 = jnp.pad(Wg, ((0, 0), (0, D - 4)))
    bg8 = jnp.pad(bg.reshape(1, 4), ((0, 0), (0, D - 4)))
    w18 = jnp.pad(W1, ((0, D - 4), (0, 0)))

    sc_degree, sc_message = _build_sc_kernels()
    deg_parts = sc_degree(ei, zeros, ones)

    y = pl.pallas_call(
        _tc_normalize_body,
        grid=(TC_GRID,),
        in_specs=[
            pl.BlockSpec((2, TC_BLK, D), lambda i: (0, i, 0)),
            pl.BlockSpec((TC_BLK, 6), lambda i: (i, 0)),
            pl.BlockSpec((6, D), lambda i: (0, 0)),
        ],
        out_specs=pl.BlockSpec((TC_BLK, D), lambda i: (i, 0)),
        out_shape=jax.ShapeDtypeStruct((N_PAD, D), jnp.float32),
    )(deg_parts, x_pad, wg8)

    acc_parts = sc_message(ei, y, zeros)

    out = pl.pallas_call(
        _tc_finalize_body,
        grid=(TC_GRID,),
        in_specs=[
            pl.BlockSpec((2, TC_BLK, D), lambda i: (0, i, 0)),
            pl.BlockSpec((TC_BLK, D), lambda i: (i, 0)),
            pl.BlockSpec((2, TC_BLK, D), lambda i: (0, i, 0)),
            pl.BlockSpec((1, D), lambda i: (0, 0)),
            pl.BlockSpec((D, 256), lambda i: (0, 0)),
            pl.BlockSpec((1, 256), lambda i: (0, 0)),
            pl.BlockSpec((256, 128), lambda i: (0, 0)),
            pl.BlockSpec((1, 128), lambda i: (0, 0)),
        ],
        out_specs=pl.BlockSpec((1, 128), lambda i: (0, 0)),
        out_shape=jax.ShapeDtypeStruct((1, 128), jnp.float32),
        scratch_shapes=[pltpu.VMEM((8, 128), jnp.float32)],
    )(acc_parts, y, deg_parts, bg8, w18, b1.reshape(1, 256),
      W3, b3.reshape(1, 128))

    return out
